# baseline (speedup 1.0000x reference)
import functools

import jax
import jax.numpy as jnp
from jax import lax
from jax.experimental import pallas as pl
from jax.experimental.pallas import tpu as pltpu
from jax.experimental.pallas import tpu_sc as plsc

D = 128
N_GRID = 10000
N_MESH = 10000
E = 320000

_NC = 2
_NS = 16
_NW = _NC * _NS
_EPW = E // _NW
_CH = 80
_NCHUNK = _EPW // _CH
_NPAD = 10240
_STRIPE = _NPAD // _NS


def _silu(x):
    return x * jax.nn.sigmoid(x)


def _ln(y, g, bt):
    mu = jnp.mean(y, axis=-1, keepdims=True)
    yc = y - mu
    var = jnp.mean(yc * yc, axis=-1, keepdims=True)
    return yc * lax.rsqrt(var + 1e-5) * g + bt


def _full_spec(a):
    nd = a.ndim
    return pl.BlockSpec(a.shape, lambda i, _n=nd: (0,) * _n)


def _row_spec(rows, cols):
    return pl.BlockSpec((rows, cols), lambda i: (i, 0))


def _gm_body(xg, xm,
             gw1, gb1, gw2, gb2, gg, gbt, ws,
             mw1, mb1, mw2, mb2, mg, mbt, wd,
             g_ref, gs_ref, m_ref, md_ref):
    h = _silu(jnp.dot(xg[...], gw1[...], preferred_element_type=jnp.float32)
              + gb1[...])
    g = _ln(jnp.dot(h, gw2[...], preferred_element_type=jnp.float32)
            + gb2[...], gg[...], gbt[...])
    g_ref[...] = g
    gs_ref[...] = jnp.dot(g, ws[...], preferred_element_type=jnp.float32)
    hm = _silu(jnp.dot(xm[...], mw1[...], preferred_element_type=jnp.float32)
               + mb1[...])
    m = _ln(jnp.dot(hm, mw2[...], preferred_element_type=jnp.float32)
            + mb2[...], mg[...], mbt[...])
    m_ref[...] = m
    md_ref[...] = jnp.dot(m, wd[...], preferred_element_type=jnp.float32)


def _gout_body(g, nw1, nb1, nw2, nb2, ng, nbt, gout_ref):
    h2 = _silu(jnp.dot(g[...], nw1[...], preferred_element_type=jnp.float32)
               + nb1[...])
    y2 = jnp.dot(h2, nw2[...], preferred_element_type=jnp.float32) + nb2[...]
    gout_ref[...] = g[...] + _ln(y2, ng[...], nbt[...])


def _edge_body(eft, sump, ew1, eb1, ew2, eb2, eg, ebt,
               we, ib1, iw2, ib2, ig, ibt, eout_ref, enew_ref):
    h0pre = jax.lax.dot_general(
        eft[...], ew1[...], (((0,), (0,)), ((), ())),
        preferred_element_type=jnp.float32)
    h0 = _silu(h0pre + eb1[...])
    e = _ln(jnp.dot(h0, ew2[...], preferred_element_type=jnp.float32)
            + eb2[...], eg[...], ebt[...])
    pre = (jnp.dot(e, we[...], preferred_element_type=jnp.float32) + ib1[...]
           + sump[...])
    h = _silu(pre)
    en = _ln(jnp.dot(h, iw2[...], preferred_element_type=jnp.float32)
             + ib2[...], ig[...], ibt[...])
    enew_ref[...] = en
    eout_ref[...] = e + en


def _node_body(m, pa0, pa1, pb0, pb1, wa, wm, b1, w2, b2, gg, bt, mout_ref):
    agg = pa0[0] + pa1[0] + pb0[0] + pb1[0]
    h = _silu(jnp.dot(agg, wa[...], preferred_element_type=jnp.float32)
              + jnp.dot(m[...], wm[...], preferred_element_type=jnp.float32)
              + b1[...])
    mn = _ln(jnp.dot(h, w2[...], preferred_element_type=jnp.float32) + b2[...],
             gg[...], bt[...])
    mout_ref[...] = m[...] + mn


def _run_rows(body, grid_n, row_block, ins, outs, n_blocked=1):
    out_shape = tuple(jax.ShapeDtypeStruct((grid_n * row_block, c), dt)
                      for c, dt in outs)
    in_specs = [_row_spec(row_block, a.shape[-1]) if k < n_blocked
                else _full_spec(a) for k, a in enumerate(ins)]
    out_specs = tuple(_row_spec(row_block, c) for c, _ in outs)
    one = len(outs) == 1
    return pl.pallas_call(
        body,
        grid=(grid_n,),
        in_specs=in_specs,
        out_specs=out_specs[0] if one else out_specs,
        out_shape=out_shape[0] if one else out_shape,
    )(*ins)


@functools.lru_cache(maxsize=None)
def _build_sc_gather(nch):
    mesh = plsc.VectorSubcoreMesh(core_axis_name="c", subcore_axis_name="s")
    P = 4
    NRING = (nch // P) * P
    epw = nch * _CH

    @functools.partial(
        pl.kernel, mesh=mesh,
        out_type=jax.ShapeDtypeStruct((_NW * epw, D), jnp.float32),
        scratch_types=[pltpu.VMEM((nch, _CH), jnp.int32),
                       pltpu.VMEM((nch, _CH), jnp.int32)]
                      + [pltpu.VMEM((_CH, D), jnp.float32)] * (2 * P)
                      + [pltpu.SemaphoreType.DMA] * (3 * P),
    )
    def sc_gather(gs_hbm, md_hbm, idx0_hbm, idx1_hbm, sum_hbm,
                  idx0_v, idx1_v, *bufsems):
        ra = bufsems[0:P]
        rb = bufsems[P:2 * P]
        sga = bufsems[2 * P:3 * P]
        sgb = bufsems[3 * P:4 * P]
        sw = bufsems[4 * P:5 * P]
        wid = lax.axis_index("s") * _NC + lax.axis_index("c")
        pltpu.sync_copy(idx0_hbm.at[wid], idx0_v)
        pltpu.sync_copy(idx1_hbm.at[wid], idx1_v)

        def add_into(dst, src):
            def add_body(i, carry):
                for q in range(D // 16):
                    o = q * 16
                    dst[i, pl.ds(o, 16)] = (dst[i, pl.ds(o, 16)]
                                            + src[i, pl.ds(o, 16)])
                return carry
            lax.fori_loop(0, _CH, add_body, 0)

        def start(c, u):
            pltpu.async_copy(gs_hbm.at[idx0_v.at[c]], ra[u], sga[u])
            pltpu.async_copy(md_hbm.at[idx1_v.at[c]], rb[u], sgb[u])

        def wait_gather(c, u):
            pltpu.make_async_copy(gs_hbm.at[idx0_v.at[c]], ra[u],
                                  sga[u]).wait()
            pltpu.make_async_copy(md_hbm.at[idx1_v.at[c]], rb[u],
                                  sgb[u]).wait()

        for u in range(P):
            start(u, u)

        def body(k, carry):
            for u in range(P):
                c = P * k + u
                b = wid * epw + c * _CH
                wait_gather(c, u)
                add_into(ra[u], rb[u])
                pltpu.async_copy(ra[u], sum_hbm.at[pl.ds(b, _CH)], sw[u])
            for u in range(P):
                c = P * k + u
                cn = c + P
                b = wid * epw + c * _CH
                pltpu.make_async_copy(ra[u], sum_hbm.at[pl.ds(b, _CH)],
                                      sw[u]).wait()

                @pl.when(cn < NRING)
                def _():
                    start(cn, u)
            return carry

        lax.fori_loop(0, NRING // P, body, 0)
        for ct in range(NRING, nch):
            bt = wid * epw + ct * _CH
            start(ct, 0)
            wait_gather(ct, 0)
            add_into(ra[0], rb[0])
            pltpu.sync_copy(ra[0], sum_hbm.at[pl.ds(bt, _CH)])

    return sc_gather


@functools.lru_cache(maxsize=None)
def _build_sc_scatter(e0, nch):
    mesh = plsc.VectorSubcoreMesh(core_axis_name="c", subcore_axis_name="s")

    P = 3
    NRING = (nch // P) * P
    epw = nch * _CH

    @functools.partial(
        pl.kernel, mesh=mesh,
        out_type=jax.ShapeDtypeStruct((_NC, _NPAD, D), jnp.float32),
        scratch_types=[pltpu.VMEM((1, _CH), jnp.int32)] * P
                      + [pltpu.VMEM((_CH, D), jnp.float32)] * P
                      + [pltpu.VMEM_SHARED((_NPAD, D), jnp.float32)]
                      + [pltpu.SemaphoreType.DMA] * (3 * P),
    )
    def sc_scatter(enew_hbm, idx1_hbm, zeros_hbm, out_hbm, *rest):
        ibuf = rest[0:P]
        rbuf = rest[P:2 * P]
        acc_sh = rest[2 * P]
        sr = rest[2 * P + 1:3 * P + 1]
        si = rest[3 * P + 1:4 * P + 1]
        sa = rest[4 * P + 1:5 * P + 1]
        cid = lax.axis_index("c")
        sid = lax.axis_index("s")
        wid = sid * _NC + cid
        pltpu.sync_copy(zeros_hbm, acc_sh.at[pl.ds(sid * _STRIPE, _STRIPE)])
        plsc.subcore_barrier()

        def start_read(c, u):
            b = e0 + wid * epw + c * _CH
            pltpu.async_copy(enew_hbm.at[pl.ds(b, _CH)], rbuf[u], sr[u])

        def start_idx(c, u):
            pltpu.async_copy(idx1_hbm.at[wid].at[pl.ds(c, 1)], ibuf[u],
                             si[u])

        def wait_read(c, u):
            b = e0 + wid * epw + c * _CH
            pltpu.make_async_copy(enew_hbm.at[pl.ds(b, _CH)], rbuf[u],
                                  sr[u]).wait()
            pltpu.make_async_copy(idx1_hbm.at[wid].at[pl.ds(c, 1)], ibuf[u],
                                  si[u]).wait()

        for u in range(P):
            start_read(u, u)
            start_idx(u, u)

        def body(k, carry):
            for u in range(P):
                c = P * k + u
                wait_read(c, u)
                pltpu.async_copy(rbuf[u], acc_sh.at[ibuf[u].at[0]], sa[u],
                                 add=True)
            for u in range(P):
                c = P * k + u
                pltpu.make_async_copy(rbuf[u], acc_sh.at[ibuf[u].at[0]],
                                      sa[u]).wait()

                @pl.when(c + P < NRING)
                def _():
                    start_read(c + P, u)
                    start_idx(c + P, u)
            return carry

        lax.fori_loop(0, NRING // P, body, 0)
        for ct in range(NRING, nch):
            start_read(ct, 0)
            start_idx(ct, 0)
            wait_read(ct, 0)
            pltpu.sync_copy(rbuf[0], acc_sh.at[ibuf[0].at[0]], add=True)
        plsc.subcore_barrier()
        pltpu.sync_copy(acc_sh.at[pl.ds(sid * _STRIPE, _STRIPE)],
                        out_hbm.at[cid, pl.ds(sid * _STRIPE, _STRIPE)])

    return sc_scatter


def _sc_gather(gs, md, idx0, idx1, nch):
    return _build_sc_gather(nch)(gs, md, idx0, idx1)


def _sc_scatter(e_new, idx1, zeros, e0, nch):
    return _build_sc_scatter(e0, nch)(e_new, idx1, zeros)


def kernel(grid_nfeat, mesh_nfeat, edge_index, grid2mesh_efeat, params):
    p = params

    def vec(w, name):
        return w[name].reshape(1, -1)

    in_w1 = p["in_edge"]["W1"]
    w1e, w1s, w1d = in_w1[0:D], in_w1[D:2 * D], in_w1[2 * D:3 * D]
    in_node_w1 = p["in_node"]["W1"]
    wa, wm = in_node_w1[0:D], in_node_w1[D:2 * D]

    ge = p["grid_emb"]
    gn = p["grid_node"]
    me = p["mesh_emb"]
    g, gs, m, md = _run_rows(
        _gm_body, 25, 400,
        [grid_nfeat, mesh_nfeat,
         ge["W1"], vec(ge, "b1"), ge["W2"], vec(ge, "b2"),
         vec(ge, "g"), vec(ge, "bt"), w1s,
         me["W1"], vec(me, "b1"), me["W2"], vec(me, "b2"),
         vec(me, "g"), vec(me, "bt"), w1d],
        [(D, jnp.float32)] * 4, n_blocked=2)

    g_out = _run_rows(
        _gout_body, 25, 400,
        [g, gn["W1"], vec(gn, "b1"), gn["W2"], vec(gn, "b2"),
         vec(gn, "g"), vec(gn, "bt")],
        [(D, jnp.float32)])

    NCH_A = 57
    NCH_B = _NCHUNK - NCH_A
    EA = _NW * NCH_A * _CH
    idx0 = edge_index[0]
    idx1 = edge_index[1]
    i0a = idx0[:EA].reshape(_NW, NCH_A, _CH)
    i1a = idx1[:EA].reshape(_NW, NCH_A, _CH)
    i0b = idx0[EA:].reshape(_NW, NCH_B, _CH)
    i1b = idx1[EA:].reshape(_NW, NCH_B, _CH)

    sump_a = _sc_gather(gs, md, i0a, i1a, NCH_A)
    sump_b = _sc_gather(gs, md, i0b, i1b, NCH_B)

    ee = p["edge_emb"]
    ie = p["in_edge"]
    RE = 2560
    OFF = EA // RE
    eft = grid2mesh_efeat.T
    ew = [ee["W1"], vec(ee, "b1"), ee["W2"], vec(ee, "b2"),
          vec(ee, "g"), vec(ee, "bt"),
          w1e, vec(ie, "b1"), ie["W2"], vec(ie, "b2"),
          vec(ie, "g"), vec(ie, "bt")]
    eshape = (jax.ShapeDtypeStruct((E, D), jnp.float32),
              jax.ShapeDtypeStruct((E, D), jnp.float32))

    eins_a = [eft, sump_a] + ew
    e_out_a, e_new_a = pl.pallas_call(
        _edge_body,
        grid=(OFF,),
        in_specs=[pl.BlockSpec((4, RE), lambda i: (0, i)),
                  _row_spec(RE, D)] + [_full_spec(a) for a in ew],
        out_specs=(_row_spec(RE, D), _row_spec(RE, D)),
        out_shape=(jax.ShapeDtypeStruct((E, D), jnp.float32),
                   jax.ShapeDtypeStruct((EA, D), jnp.float32)),
    )(*eins_a)

    zeros = jnp.zeros((_STRIPE, D), jnp.float32)
    partials_a = _sc_scatter(e_new_a, i1a, zeros, 0, NCH_A)

    def _edge_body_b(eft, sump, *args):
        _edge_body(eft, sump, *args[:12], args[13], args[14])

    eins_b = [eft, sump_b] + ew + [e_out_a]
    e_out, e_new_b = pl.pallas_call(
        _edge_body_b,
        grid=(E // RE - OFF,),
        in_specs=[pl.BlockSpec((4, RE), lambda i: (0, i + OFF)),
                  _row_spec(RE, D)] + [_full_spec(a) for a in ew]
                 + [pl.BlockSpec(memory_space=pl.ANY)],
        out_specs=(pl.BlockSpec((RE, D), lambda i: (i + OFF, 0)),
                   _row_spec(RE, D)),
        out_shape=(jax.ShapeDtypeStruct((E, D), jnp.float32),
                   jax.ShapeDtypeStruct((E - EA, D), jnp.float32)),
        input_output_aliases={len(eins_b) - 1: 0},
    )(*eins_b)

    partials_b = _sc_scatter(e_new_b, i1b, zeros, 0, NCH_B)

    inn = p["in_node"]
    nins = [m, partials_a, partials_a, partials_b, partials_b,
            wa, wm, vec(inn, "b1"), inn["W2"],
            vec(inn, "b2"), vec(inn, "g"), vec(inn, "bt")]
    m_out = pl.pallas_call(
        _node_body,
        grid=(25,),
        in_specs=[_row_spec(400, D),
                  pl.BlockSpec((1, 400, D), lambda i: (0, i, 0)),
                  pl.BlockSpec((1, 400, D), lambda i: (1, i, 0)),
                  pl.BlockSpec((1, 400, D), lambda i: (0, i, 0)),
                  pl.BlockSpec((1, 400, D), lambda i: (1, i, 0))]
                 + [_full_spec(a) for a in nins[5:]],
        out_specs=_row_spec(400, D),
        out_shape=jax.ShapeDtypeStruct((N_MESH, D), jnp.float32),
    )(*nins)

    return (g_out, m_out, e_out)
